# baseline (speedup 1.0000x reference)
import functools

import jax
import jax.numpy as jnp
from jax.experimental import pallas as pl
from jax.experimental.pallas import tpu as pltpu

_C = 96
_KS = (3, 5, 7)
_HT = 8
_PAD = 3


def _silu(v):
    return v * jax.nn.sigmoid(v)


def _pass1_body(xc_ref, xn_ref,
                r1w_ref, r1b_ref, r2w_ref, r2b_ref,
                dw0_ref, db0_ref, pw0_ref, pb0_ref,
                dw1_ref, db1_ref, pw1_ref, pb1_ref,
                dw2_ref, db2_ref, pw2_ref, pb2_ref,
                out_ref, s1_ref, s2_ref, slab_ref):
    HT = out_ref.shape[1]
    W = out_ref.shape[2]
    C = out_ref.shape[3]
    a = jnp.concatenate([xc_ref[0], xn_ref[0, :2 * _PAD]], axis=0)
    xcen = a[_PAD:_PAD + HT, _PAD:_PAD + W, :].reshape(HT * W, C)

    r = jnp.dot(xcen, r1w_ref[...], preferred_element_type=jnp.float32)
    r = _silu(r + r1b_ref[...])
    logits = jnp.dot(r, r2w_ref[...], preferred_element_type=jnp.float32)
    logits = logits + r2b_ref[...]
    m = jnp.max(logits, axis=1, keepdims=True)
    p = jnp.exp(logits - m)
    rw = p / jnp.sum(p, axis=1, keepdims=True)

    for j in range(2 * _PAD + 1):
        slab_ref[j] = a[:, j:j + W, :]

    out = jnp.zeros((HT * W, C), jnp.float32)
    experts = ((dw0_ref, db0_ref, pw0_ref, pb0_ref),
               (dw1_ref, db1_ref, pw1_ref, pb1_ref),
               (dw2_ref, db2_ref, pw2_ref, pb2_ref))
    for e, k in enumerate(_KS):
        dwr, dbr, pwr, pbr = experts[e]
        off = _PAD - k // 2
        acc = jnp.zeros((HT, W, C), jnp.float32)
        for i in range(k):
            for j in range(k):
                tap = dwr[i * k + j, :][None, None, :]
                acc = acc + slab_ref[off + j, off + i:off + i + HT] * tap
        y = _silu(acc + dbr[...][None]).reshape(HT * W, C)
        eo = jnp.dot(y, pwr[...], preferred_element_type=jnp.float32)
        eo = eo + pbr[...]
        out = out + eo * rw[:, e:e + 1]

    out_ref[0] = out.reshape(HT, W, C)
    s1_ref[0, 0] = jnp.sum(out, axis=0, keepdims=True)
    s2_ref[0, 0] = jnp.sum(out * out, axis=0, keepdims=True)


def _pass2_body(out_ref, s1_ref, s2_ref, g_ref, b_ref, y_ref, *, n):
    s1 = jnp.sum(s1_ref[...], axis=(0, 1, 2))
    s2 = jnp.sum(s2_ref[...], axis=(0, 1, 2))
    mean = s1 / n
    var = s2 / n - mean * mean
    scale = g_ref[0] * jax.lax.rsqrt(var + 1e-5)
    shift = b_ref[0] - mean * scale
    HT, W, C = out_ref.shape[1], out_ref.shape[2], out_ref.shape[3]
    o = out_ref[0].reshape(HT * W, C)
    y = _silu(o * scale[None, :] + shift[None, :])
    y_ref[0] = y.T


def kernel(x, r1_w, r1_b, r2_w, r2_b,
           dw0_w, dw0_b, pw0_w, pw0_b,
           dw1_w, dw1_b, pw1_w, pw1_b,
           dw2_w, dw2_b, pw2_w, pw2_b,
           bn_gamma, bn_beta):
    B, C, H, W = x.shape
    HT = _HT
    T = H // HT
    HP = (T + 1) * HT
    Wp = W + 2 * _PAD

    xt = jnp.transpose(x, (0, 2, 3, 1))
    xp = jnp.pad(xt, ((0, 0), (_PAD, HP - H - _PAD), (_PAD, _PAD), (0, 0)))

    wargs = (
        r1_w.T, r1_b[None], r2_w.T, r2_b[None],
        dw0_w.reshape(C, -1).T, dw0_b[None], pw0_w.T, pw0_b[None],
        dw1_w.reshape(C, -1).T, dw1_b[None], pw1_w.T, pw1_b[None],
        dw2_w.reshape(C, -1).T, dw2_b[None], pw2_w.T, pw2_b[None],
    )

    def full_spec(arr):
        nd = arr.ndim
        return pl.BlockSpec(arr.shape, lambda b, t, _nd=nd: (0,) * _nd)

    xblk = pl.BlockSpec((1, HT, Wp, C), lambda b, t: (b, t, 0, 0))
    xblk_next = pl.BlockSpec((1, HT, Wp, C), lambda b, t: (b, t + 1, 0, 0))

    out, s1, s2 = pl.pallas_call(
        _pass1_body,
        out_shape=(
            jax.ShapeDtypeStruct((B, H, W, C), jnp.float32),
            jax.ShapeDtypeStruct((B, T, 1, C), jnp.float32),
            jax.ShapeDtypeStruct((B, T, 1, C), jnp.float32),
        ),
        grid=(B, T),
        in_specs=[xblk, xblk_next] + [full_spec(w) for w in wargs],
        out_specs=(
            pl.BlockSpec((1, HT, W, C), lambda b, t: (b, t, 0, 0)),
            pl.BlockSpec((1, 1, 1, C), lambda b, t: (b, t, 0, 0)),
            pl.BlockSpec((1, 1, 1, C), lambda b, t: (b, t, 0, 0)),
        ),
        scratch_shapes=[
            pltpu.VMEM((2 * _PAD + 1, HT + 2 * _PAD, W, C), jnp.float32),
        ],
    )(xp, xp, *wargs)

    n = float(B * H * W)
    y = pl.pallas_call(
        functools.partial(_pass2_body, n=n),
        out_shape=jax.ShapeDtypeStruct((B, C, H * W), jnp.float32),
        grid=(B, T),
        in_specs=[
            pl.BlockSpec((1, HT, W, C), lambda b, t: (b, t, 0, 0)),
            full_spec(s1),
            full_spec(s2),
            pl.BlockSpec((1, C), lambda b, t: (0, 0)),
            pl.BlockSpec((1, C), lambda b, t: (0, 0)),
        ],
        out_specs=pl.BlockSpec((1, C, HT * W), lambda b, t: (b, 0, t)),
    )(out, s1, s2, bn_gamma[None], bn_beta[None])

    return y.reshape(B, C, H, W)


# speedup vs baseline: 1.2141x; 1.2141x over previous
import functools

import jax
import jax.numpy as jnp
from jax.experimental import pallas as pl
from jax.experimental.pallas import tpu as pltpu

_C = 96
_KS = (3, 5, 7)
_HT = 8
_PAD = 3


def _silu(v):
    return v * jax.nn.sigmoid(v)


def _pass1_body(xp_ref, xc_ref, xn_ref,
                r1w_ref, r1b_ref, r2w_ref, r2b_ref,
                dw0_ref, db0_ref, pw0_ref, pb0_ref,
                dw1_ref, db1_ref, pw1_ref, pb1_ref,
                dw2_ref, db2_ref, pw2_ref, pb2_ref,
                out_ref, s1_ref, s2_ref, slab_ref, *, npix):
    HT = out_ref.shape[1]
    W = out_ref.shape[2]
    C = out_ref.shape[3]
    t = pl.program_id(1)
    cat = jnp.concatenate([xp_ref[0, 0], xc_ref[0, 0], xn_ref[0, 0]], axis=1)
    gp = (jax.lax.broadcasted_iota(jnp.int32, (1, 3 * HT * W), 1)
          + (t - 1) * HT * W)
    cat = jnp.where((gp >= 0) & (gp < npix), cat, 0.0)
    rows = cat.T.reshape(3 * HT, W, C)
    a = jnp.pad(rows[HT - _PAD:2 * HT + _PAD], ((0, 0), (_PAD, _PAD), (0, 0)))
    xcen = a[_PAD:_PAD + HT, _PAD:_PAD + W, :].reshape(HT * W, C)

    r = jnp.dot(xcen, r1w_ref[...], preferred_element_type=jnp.float32)
    r = _silu(r + r1b_ref[...])
    logits = jnp.dot(r, r2w_ref[...], preferred_element_type=jnp.float32)
    logits = logits + r2b_ref[...]
    m = jnp.max(logits, axis=1, keepdims=True)
    p = jnp.exp(logits - m)
    rw = p / jnp.sum(p, axis=1, keepdims=True)

    for j in range(2 * _PAD + 1):
        slab_ref[j] = a[:, j:j + W, :]

    out = jnp.zeros((HT * W, C), jnp.float32)
    experts = ((dw0_ref, db0_ref, pw0_ref, pb0_ref),
               (dw1_ref, db1_ref, pw1_ref, pb1_ref),
               (dw2_ref, db2_ref, pw2_ref, pb2_ref))
    for e, k in enumerate(_KS):
        dwr, dbr, pwr, pbr = experts[e]
        off = _PAD - k // 2
        acc = jnp.zeros((HT, W, C), jnp.float32)
        for i in range(k):
            for j in range(k):
                tap = dwr[i * k + j, :][None, None, :]
                acc = acc + slab_ref[off + j, off + i:off + i + HT] * tap
        y = _silu(acc + dbr[...][None]).reshape(HT * W, C)
        eo = jnp.dot(y, pwr[...], preferred_element_type=jnp.float32)
        eo = eo + pbr[...]
        out = out + eo * rw[:, e:e + 1]

    out_ref[0] = out.reshape(HT, W, C)
    s1_ref[0, 0] = jnp.sum(out, axis=0, keepdims=True)
    s2_ref[0, 0] = jnp.sum(out * out, axis=0, keepdims=True)


def _pass2_body(out_ref, s1_ref, s2_ref, g_ref, b_ref, y_ref, *, n):
    s1 = jnp.sum(s1_ref[...], axis=(0, 1, 2))
    s2 = jnp.sum(s2_ref[...], axis=(0, 1, 2))
    mean = s1 / n
    var = s2 / n - mean * mean
    scale = g_ref[0] * jax.lax.rsqrt(var + 1e-5)
    shift = b_ref[0] - mean * scale
    HT, W, C = out_ref.shape[1], out_ref.shape[2], out_ref.shape[3]
    o = out_ref[0].reshape(HT * W, C)
    y = _silu(o * scale[None, :] + shift[None, :])
    y_ref[0] = y.T


def kernel(x, r1_w, r1_b, r2_w, r2_b,
           dw0_w, dw0_b, pw0_w, pw0_b,
           dw1_w, dw1_b, pw1_w, pw1_b,
           dw2_w, dw2_b, pw2_w, pw2_b,
           bn_gamma, bn_beta):
    B, C, H, W = x.shape
    HT = _HT
    T = H // HT

    xf = x.reshape(B, 1, C, H * W)

    wargs = (
        r1_w.T, r1_b[None], r2_w.T, r2_b[None],
        dw0_w.reshape(C, -1).T, dw0_b[None], pw0_w.T, pw0_b[None],
        dw1_w.reshape(C, -1).T, dw1_b[None], pw1_w.T, pw1_b[None],
        dw2_w.reshape(C, -1).T, dw2_b[None], pw2_w.T, pw2_b[None],
    )

    def full_spec(arr):
        nd = arr.ndim
        return pl.BlockSpec(arr.shape, lambda b, t, _nd=nd: (0,) * _nd)

    xblk_prev = pl.BlockSpec((1, 1, C, HT * W),
                             lambda b, t: (b, 0, 0, jnp.maximum(t - 1, 0)))
    xblk = pl.BlockSpec((1, 1, C, HT * W), lambda b, t: (b, 0, 0, t))
    xblk_next = pl.BlockSpec((1, 1, C, HT * W),
                             lambda b, t: (b, 0, 0, jnp.minimum(t + 1, T - 1)))

    out, s1, s2 = pl.pallas_call(
        functools.partial(_pass1_body, npix=H * W),
        out_shape=(
            jax.ShapeDtypeStruct((B, H, W, C), jnp.float32),
            jax.ShapeDtypeStruct((B, T, 1, C), jnp.float32),
            jax.ShapeDtypeStruct((B, T, 1, C), jnp.float32),
        ),
        grid=(B, T),
        in_specs=[xblk_prev, xblk, xblk_next] + [full_spec(w) for w in wargs],
        out_specs=(
            pl.BlockSpec((1, HT, W, C), lambda b, t: (b, t, 0, 0)),
            pl.BlockSpec((1, 1, 1, C), lambda b, t: (b, t, 0, 0)),
            pl.BlockSpec((1, 1, 1, C), lambda b, t: (b, t, 0, 0)),
        ),
        scratch_shapes=[
            pltpu.VMEM((2 * _PAD + 1, HT + 2 * _PAD, W, C), jnp.float32),
        ],
    )(xf, xf, xf, *wargs)

    n = float(B * H * W)
    y = pl.pallas_call(
        functools.partial(_pass2_body, n=n),
        out_shape=jax.ShapeDtypeStruct((B, C, H * W), jnp.float32),
        grid=(B, T),
        in_specs=[
            pl.BlockSpec((1, HT, W, C), lambda b, t: (b, t, 0, 0)),
            full_spec(s1),
            full_spec(s2),
            pl.BlockSpec((1, C), lambda b, t: (0, 0)),
            pl.BlockSpec((1, C), lambda b, t: (0, 0)),
        ],
        out_specs=pl.BlockSpec((1, C, HT * W), lambda b, t: (b, 0, t)),
    )(out, s1, s2, bn_gamma[None], bn_beta[None])

    return y.reshape(B, C, H, W)


# speedup vs baseline: 1.2348x; 1.0171x over previous
import functools

import jax
import jax.numpy as jnp
from jax.experimental import pallas as pl
from jax.experimental.pallas import tpu as pltpu

_C = 96
_KS = (3, 5, 7)
_HT = 8
_PAD = 3


def _silu(v):
    return v * jax.nn.sigmoid(v)


def _pass1_body(xin_ref,
                r1w_ref, r1b_ref, r2w_ref, r2b_ref,
                dw0_ref, db0_ref, pw0_ref, pb0_ref,
                dw1_ref, db1_ref, pw1_ref, pb1_ref,
                dw2_ref, db2_ref, pw2_ref, pb2_ref,
                out_ref, s1_ref, s2_ref, chunks_ref, slab_ref, *, nrows):
    HT = out_ref.shape[1]
    W = out_ref.shape[2]
    C = out_ref.shape[3]
    s = pl.program_id(1)

    chunks_ref[0] = chunks_ref[1]
    chunks_ref[1] = chunks_ref[2]
    chunks_ref[2] = xin_ref[0, 0].T.reshape(HT, W, C)

    @pl.when(s >= 1)
    def _compute():
        t = s - 1
        a24 = jnp.concatenate(
            [chunks_ref[0], chunks_ref[1], chunks_ref[2]], axis=0)
        win = a24[HT - _PAD:2 * HT + _PAD]
        g = (jax.lax.broadcasted_iota(jnp.int32, (HT + 2 * _PAD, W, 1), 0)
             + t * HT - _PAD)
        win = jnp.where((g >= 0) & (g < nrows), win, 0.0)
        a = jnp.pad(win, ((0, 0), (_PAD, _PAD), (0, 0)))
        xcen = a[_PAD:_PAD + HT, _PAD:_PAD + W, :].reshape(HT * W, C)

        r = jnp.dot(xcen, r1w_ref[...], preferred_element_type=jnp.float32)
        r = _silu(r + r1b_ref[...])
        logits = jnp.dot(r, r2w_ref[...], preferred_element_type=jnp.float32)
        logits = logits + r2b_ref[...]
        m = jnp.max(logits, axis=1, keepdims=True)
        p = jnp.exp(logits - m)
        rw = p / jnp.sum(p, axis=1, keepdims=True)

        for j in range(2 * _PAD + 1):
            slab_ref[j] = a[:, j:j + W, :]

        out = jnp.zeros((HT * W, C), jnp.float32)
        experts = ((dw0_ref, db0_ref, pw0_ref, pb0_ref),
                   (dw1_ref, db1_ref, pw1_ref, pb1_ref),
                   (dw2_ref, db2_ref, pw2_ref, pb2_ref))
        for e, k in enumerate(_KS):
            dwr, dbr, pwr, pbr = experts[e]
            off = _PAD - k // 2
            acc = jnp.zeros((HT, W, C), jnp.float32)
            for i in range(k):
                for j in range(k):
                    tap = dwr[i * k + j, :][None, None, :]
                    acc = acc + slab_ref[off + j, off + i:off + i + HT] * tap
            y = _silu(acc + dbr[...][None]).reshape(HT * W, C)
            eo = jnp.dot(y, pwr[...], preferred_element_type=jnp.float32)
            eo = eo + pbr[...]
            out = out + eo * rw[:, e:e + 1]

        out_ref[0] = out.reshape(HT, W, C).astype(jnp.bfloat16)
        s1_ref[0, 0] = jnp.sum(out, axis=0, keepdims=True)
        s2_ref[0, 0] = jnp.sum(out * out, axis=0, keepdims=True)


def _pass2_body(out_ref, s1_ref, s2_ref, g_ref, b_ref, y_ref, *, n):
    s1 = jnp.sum(s1_ref[...], axis=(0, 1, 2))
    s2 = jnp.sum(s2_ref[...], axis=(0, 1, 2))
    mean = s1 / n
    var = s2 / n - mean * mean
    scale = g_ref[0] * jax.lax.rsqrt(var + 1e-5)
    shift = b_ref[0] - mean * scale
    HT, W, C = out_ref.shape[1], out_ref.shape[2], out_ref.shape[3]
    o = out_ref[0].astype(jnp.float32).reshape(HT * W, C)
    y = _silu(o * scale[None, :] + shift[None, :])
    y_ref[0] = y.T


def kernel(x, r1_w, r1_b, r2_w, r2_b,
           dw0_w, dw0_b, pw0_w, pw0_b,
           dw1_w, dw1_b, pw1_w, pw1_b,
           dw2_w, dw2_b, pw2_w, pw2_b,
           bn_gamma, bn_beta):
    B, C, H, W = x.shape
    HT = _HT
    T = H // HT

    xf = x.reshape(B, 1, C, H * W)

    wargs = (
        r1_w.T, r1_b[None], r2_w.T, r2_b[None],
        dw0_w.reshape(C, -1).T, dw0_b[None], pw0_w.T, pw0_b[None],
        dw1_w.reshape(C, -1).T, dw1_b[None], pw1_w.T, pw1_b[None],
        dw2_w.reshape(C, -1).T, dw2_b[None], pw2_w.T, pw2_b[None],
    )

    def full_spec(arr):
        nd = arr.ndim
        return pl.BlockSpec(arr.shape, lambda b, t, _nd=nd: (0,) * _nd)

    xblk = pl.BlockSpec((1, 1, C, HT * W),
                        lambda b, s: (b, 0, 0, jnp.minimum(s, T - 1)))

    out, s1, s2 = pl.pallas_call(
        functools.partial(_pass1_body, nrows=H),
        out_shape=(
            jax.ShapeDtypeStruct((B, H, W, C), jnp.bfloat16),
            jax.ShapeDtypeStruct((B, T, 1, C), jnp.float32),
            jax.ShapeDtypeStruct((B, T, 1, C), jnp.float32),
        ),
        grid=(B, T + 1),
        in_specs=[xblk] + [full_spec(w) for w in wargs],
        out_specs=(
            pl.BlockSpec((1, HT, W, C),
                         lambda b, s: (b, jnp.maximum(s - 1, 0), 0, 0)),
            pl.BlockSpec((1, 1, 1, C),
                         lambda b, s: (b, jnp.maximum(s - 1, 0), 0, 0)),
            pl.BlockSpec((1, 1, 1, C),
                         lambda b, s: (b, jnp.maximum(s - 1, 0), 0, 0)),
        ),
        scratch_shapes=[
            pltpu.VMEM((3, HT, W, C), jnp.float32),
            pltpu.VMEM((2 * _PAD + 1, HT + 2 * _PAD, W, C), jnp.float32),
        ],
    )(xf, *wargs)

    n = float(B * H * W)
    y = pl.pallas_call(
        functools.partial(_pass2_body, n=n),
        out_shape=jax.ShapeDtypeStruct((B, C, H * W), jnp.float32),
        grid=(B, T),
        in_specs=[
            pl.BlockSpec((1, HT, W, C), lambda b, t: (b, t, 0, 0)),
            full_spec(s1),
            full_spec(s2),
            pl.BlockSpec((1, C), lambda b, t: (0, 0)),
            pl.BlockSpec((1, C), lambda b, t: (0, 0)),
        ],
        out_specs=pl.BlockSpec((1, C, HT * W), lambda b, t: (b, 0, t)),
    )(out, s1, s2, bn_gamma[None], bn_beta[None])

    return y.reshape(B, C, H, W)
